# initial kernel scaffold (unmeasured)
import jax
import jax.numpy as jnp
from jax import lax
from jax.experimental import pallas as pl
from jax.experimental.pallas import tpu as pltpu

M = 4096
D = 4096
EPS = 1e-6


def _exchange(partial):

    def body(p_ref, recv_ref, send_sem, recv_sem):
        my_x = lax.axis_index("x")
        my_y = lax.axis_index("y")
        my_z = lax.axis_index("z")
        nbr = (my_x, 1 - my_y, my_z)

        barrier_sem = pltpu.get_barrier_semaphore()
        pl.semaphore_signal(
            barrier_sem, inc=1, device_id=nbr,
            device_id_type=pl.DeviceIdType.MESH,
        )
        pl.semaphore_wait(barrier_sem, 1)

        rdma = pltpu.make_async_remote_copy(
            src_ref=p_ref.at[0],
            dst_ref=recv_ref,
            send_sem=send_sem,
            recv_sem=recv_sem,
            device_id=nbr,
            device_id_type=pl.DeviceIdType.MESH,
        )
        rdma.start()
        rdma.wait()

    return pl.pallas_call(
        body,
        out_shape=jax.ShapeDtypeStruct((M, D), partial.dtype),
        in_specs=[pl.BlockSpec(memory_space=pltpu.ANY)],
        out_specs=pl.BlockSpec(memory_space=pltpu.ANY),
        scratch_shapes=[pltpu.SemaphoreType.DMA, pltpu.SemaphoreType.DMA],
        compiler_params=pltpu.CompilerParams(collective_id=0),
    )(partial)


def _ln(partial, recv, resid, gamma2d):
    TILE = 256

    def body(p_ref, r_ref, res_ref, g_ref, o_ref):
        y = p_ref[0] + r_ref[...] + res_ref[...]
        ms = jnp.mean(y * y, axis=-1, keepdims=True)
        o_ref[...] = y * lax.rsqrt(ms + EPS) * g_ref[...]

    return pl.pallas_call(
        body,
        grid=(M // TILE,),
        in_specs=[
            pl.BlockSpec((1, TILE, D), lambda i: (0, i, 0)),
            pl.BlockSpec((TILE, D), lambda i: (i, 0)),
            pl.BlockSpec((TILE, D), lambda i: (i, 0)),
            pl.BlockSpec((1, D), lambda i: (0, 0)),
        ],
        out_specs=pl.BlockSpec((TILE, D), lambda i: (i, 0)),
        out_shape=jax.ShapeDtypeStruct((M, D), jnp.float32),
    )(partial, recv, resid, gamma2d)


def kernel(partial, resid, gamma):
    recv = _exchange(partial)
    return _ln(partial, recv, resid, gamma.reshape(1, D))


# baseline (device time: 807418 ns/iter reference)
import jax
import jax.numpy as jnp
from jax import lax
from jax.experimental import pallas as pl
from jax.experimental.pallas import tpu as pltpu

M = 4096
D = 4096
EPS = 1e-6


def _exchange(partial):

    def body(p_ref, recv_ref, send_sem, recv_sem):
        my_x = lax.axis_index("x")
        my_y = lax.axis_index("y")
        my_z = lax.axis_index("z")
        nbr = (my_x, 1 - my_y, my_z)

        barrier_sem = pltpu.get_barrier_semaphore()
        pl.semaphore_signal(
            barrier_sem, inc=1, device_id=nbr,
            device_id_type=pl.DeviceIdType.MESH,
        )
        pl.semaphore_wait(barrier_sem, 1)

        rdma = pltpu.make_async_remote_copy(
            src_ref=p_ref.at[0],
            dst_ref=recv_ref,
            send_sem=send_sem,
            recv_sem=recv_sem,
            device_id=nbr,
            device_id_type=pl.DeviceIdType.MESH,
        )
        rdma.start()
        rdma.wait()

    return pl.pallas_call(
        body,
        out_shape=jax.ShapeDtypeStruct((M, D), partial.dtype),
        in_specs=[pl.BlockSpec(memory_space=pl.ANY)],
        out_specs=pl.BlockSpec(memory_space=pl.ANY),
        scratch_shapes=[pltpu.SemaphoreType.DMA, pltpu.SemaphoreType.DMA],
        compiler_params=pltpu.CompilerParams(collective_id=0),
    )(partial)


def _ln(partial, recv, resid, gamma2d):
    TILE = 128

    def body(p_ref, r_ref, res_ref, g_ref, o_ref):
        y = p_ref[0] + r_ref[...] + res_ref[...]
        ms = jnp.mean(y * y, axis=-1, keepdims=True)
        o_ref[...] = y * lax.rsqrt(ms + EPS) * g_ref[...]

    return pl.pallas_call(
        body,
        grid=(M // TILE,),
        in_specs=[
            pl.BlockSpec((1, TILE, D), lambda i: (0, i, 0)),
            pl.BlockSpec((TILE, D), lambda i: (i, 0)),
            pl.BlockSpec((TILE, D), lambda i: (i, 0)),
            pl.BlockSpec((1, D), lambda i: (0, 0)),
        ],
        out_specs=pl.BlockSpec((TILE, D), lambda i: (i, 0)),
        out_shape=jax.ShapeDtypeStruct((M, D), jnp.float32),
    )(partial, recv, resid, gamma2d)


def kernel(partial, resid, gamma):
    recv = _exchange(partial)
    return _ln(partial, recv, resid, gamma.reshape(1, D))


# device time: 299817 ns/iter; 2.6930x vs baseline; 2.6930x over previous
import jax
import jax.numpy as jnp
from jax import lax
from jax.experimental import pallas as pl
from jax.experimental.pallas import tpu as pltpu

M = 4096
D = 4096
CH = M // 4
H = CH // 2
TILE = 128
EPS = 1e-6


def _to_bf16_chunk(partial, c):

    def body(c_ref, p_ref, o_ref):
        o_ref[...] = p_ref[0].astype(jnp.bfloat16)

    grid_spec = pltpu.PrefetchScalarGridSpec(
        num_scalar_prefetch=1,
        grid=(CH // TILE,),
        in_specs=[
            pl.BlockSpec((1, TILE, D), lambda i, c: (0, c[0] * (CH // TILE) + i, 0)),
        ],
        out_specs=pl.BlockSpec((TILE, D), lambda i, c: (i, 0)),
    )
    return pl.pallas_call(
        body,
        grid_spec=grid_spec,
        out_shape=jax.ShapeDtypeStruct((CH, D), jnp.bfloat16),
    )(c, partial)


def _exchange_y(pbf_c):

    def body(src_ref, recv_ref, send_sem, recv_sem):
        my_x = lax.axis_index("x")
        my_y = lax.axis_index("y")
        my_z = lax.axis_index("z")
        nbr = (my_x, 1 - my_y, my_z)

        barrier_sem = pltpu.get_barrier_semaphore()
        pl.semaphore_signal(
            barrier_sem, inc=1, device_id=nbr,
            device_id_type=pl.DeviceIdType.MESH,
        )
        pl.semaphore_wait(barrier_sem, 1)

        rdma = pltpu.make_async_remote_copy(
            src_ref=src_ref,
            dst_ref=recv_ref,
            send_sem=send_sem,
            recv_sem=recv_sem,
            device_id=nbr,
            device_id_type=pl.DeviceIdType.MESH,
        )
        rdma.start()
        rdma.wait()

    return pl.pallas_call(
        body,
        out_shape=jax.ShapeDtypeStruct((CH, D), jnp.bfloat16),
        in_specs=[pl.BlockSpec(memory_space=pl.ANY)],
        out_specs=pl.BlockSpec(memory_space=pl.ANY),
        scratch_shapes=[pltpu.SemaphoreType.DMA, pltpu.SemaphoreType.DMA],
        compiler_params=pltpu.CompilerParams(collective_id=0),
    )(pbf_c)


def _norm_chunk(partial, recv_y, resid, gamma2d, c):

    def body(c_ref, p_ref, r_ref, res_ref, g_ref, o_ref):
        y = p_ref[0] + r_ref[...].astype(jnp.float32) + res_ref[...]
        ms = jnp.mean(y * y, axis=-1, keepdims=True)
        o_ref[...] = (y * lax.rsqrt(ms + EPS) * g_ref[...]).astype(jnp.bfloat16)

    nblk = CH // TILE
    grid_spec = pltpu.PrefetchScalarGridSpec(
        num_scalar_prefetch=1,
        grid=(nblk,),
        in_specs=[
            pl.BlockSpec((1, TILE, D), lambda i, c: (0, c[0] * nblk + i, 0)),
            pl.BlockSpec((TILE, D), lambda i, c: (i, 0)),
            pl.BlockSpec((TILE, D), lambda i, c: (c[0] * nblk + i, 0)),
            pl.BlockSpec((1, D), lambda i, c: (0, 0)),
        ],
        out_specs=pl.BlockSpec((TILE, D), lambda i, c: (c[0] * nblk + i, 0)),
    )
    return pl.pallas_call(
        body,
        grid_spec=grid_spec,
        out_shape=jax.ShapeDtypeStruct((M, D), jnp.bfloat16),
    )(c, partial, recv_y, resid, gamma2d)


def _gather_xz(full_bf):

    def body(in_ref, out_ref, s_x, s_z, s_fx, s_fz, r_x, r_z, r_fx, r_fz):
        del in_ref
        my_x = lax.axis_index("x")
        my_z = lax.axis_index("z")
        nbr_x = (1 - my_x, lax.axis_index("y"), my_z)
        nbr_z = (my_x, lax.axis_index("y"), 1 - my_z)

        c0 = (my_x * 2 + my_z) * CH
        cx0 = ((1 - my_x) * 2 + my_z) * CH
        cz0 = (my_x * 2 + (1 - my_z)) * CH
        cd0 = ((1 - my_x) * 2 + (1 - my_z)) * CH

        barrier_sem = pltpu.get_barrier_semaphore()
        for nbr in (nbr_x, nbr_z):
            pl.semaphore_signal(
                barrier_sem, inc=1, device_id=nbr,
                device_id_type=pl.DeviceIdType.MESH,
            )
        pl.semaphore_wait(barrier_sem, 2)

        def copy(rows0, nrows, sem_s, sem_r, nbr):
            return pltpu.make_async_remote_copy(
                src_ref=out_ref.at[pl.ds(rows0, nrows)],
                dst_ref=out_ref.at[pl.ds(rows0, nrows)],
                send_sem=sem_s,
                recv_sem=sem_r,
                device_id=nbr,
                device_id_type=pl.DeviceIdType.MESH,
            )

        send_x = copy(c0, CH, s_x, r_x, nbr_x)
        send_z = copy(c0, CH, s_z, r_z, nbr_z)
        send_x.start()
        send_z.start()

        recv_x = copy(cx0, CH, s_x, r_x, nbr_x)
        recv_z = copy(cz0, CH, s_z, r_z, nbr_z)

        recv_z.wait_recv()
        fwd_x = copy(cz0, H, s_fx, r_fx, nbr_x)
        fwd_x.start()
        recv_x.wait_recv()
        fwd_z = copy(cx0 + H, H, s_fz, r_fz, nbr_z)
        fwd_z.start()

        recv_fx = copy(cd0, H, s_fx, r_fx, nbr_x)
        recv_fz = copy(cd0 + H, H, s_fz, r_fz, nbr_z)
        recv_fx.wait_recv()
        recv_fz.wait_recv()

        send_x.wait_send()
        send_z.wait_send()
        fwd_x.wait_send()
        fwd_z.wait_send()

    return pl.pallas_call(
        body,
        out_shape=jax.ShapeDtypeStruct((M, D), jnp.bfloat16),
        in_specs=[pl.BlockSpec(memory_space=pl.ANY)],
        out_specs=pl.BlockSpec(memory_space=pl.ANY),
        scratch_shapes=[pltpu.SemaphoreType.DMA] * 8,
        input_output_aliases={0: 0},
        compiler_params=pltpu.CompilerParams(collective_id=1),
    )(full_bf)


def _to_f32(full_bf):
    def body(i_ref, o_ref):
        o_ref[...] = i_ref[...].astype(jnp.float32)

    return pl.pallas_call(
        body,
        grid=(M // TILE,),
        in_specs=[pl.BlockSpec((TILE, D), lambda i: (i, 0))],
        out_specs=pl.BlockSpec((TILE, D), lambda i: (i, 0)),
        out_shape=jax.ShapeDtypeStruct((M, D), jnp.float32),
    )(full_bf)


def kernel(partial, resid, gamma):
    my_x = lax.axis_index("x")
    my_z = lax.axis_index("z")
    c = (my_x * 2 + my_z).astype(jnp.int32).reshape((1,))

    pbf_c = _to_bf16_chunk(partial, c)
    recv_y = _exchange_y(pbf_c)
    full_bf = _norm_chunk(partial, recv_y, resid, gamma.reshape(1, D), c)
    gathered = _gather_xz(full_bf)
    return _to_f32(gathered)


# device time: 251196 ns/iter; 3.2143x vs baseline; 1.1936x over previous
import jax
import jax.numpy as jnp
from jax import lax
from jax.experimental import pallas as pl
from jax.experimental.pallas import tpu as pltpu

M = 4096
D = 4096
CH = M // 4
NB = 4
BR = CH // NB
TR = BR // 2
H2 = BR // 2
TILE = 128
EPS = 1e-6

_MESH = pl.DeviceIdType.MESH


def _to_bf16_chunk(partial, c):

    def body(c_ref, p_ref, o_ref):
        o_ref[...] = p_ref[0].astype(jnp.bfloat16)

    grid_spec = pltpu.PrefetchScalarGridSpec(
        num_scalar_prefetch=1,
        grid=(CH // TILE,),
        in_specs=[
            pl.BlockSpec((1, TILE, D), lambda i, c: (0, c[0] * (CH // TILE) + i, 0)),
        ],
        out_specs=pl.BlockSpec((TILE, D), lambda i, c: (i, 0)),
    )
    return pl.pallas_call(
        body,
        grid_spec=grid_spec,
        out_shape=jax.ShapeDtypeStruct((CH, D), jnp.bfloat16),
    )(c, partial)


def _fused(partial, pbf_c, resid, gamma2d):
    def body(
        p_ref, pbf_ref, res_ref, g_ref,
        out_ref, gbuf_ref,
        recv_y, gamma_v, pstage, rstage, ntile, cvt_in, cvt_out,
        ysend, yrecv, xs, xr, zs, zr, fxs, fxr, fzs, fzr,
        gsem, psem, rsem, nsem, osem, cisem,
    ):
        my_x = lax.axis_index("x")
        my_y = lax.axis_index("y")
        my_z = lax.axis_index("z")
        nbr_y = (my_x, 1 - my_y, my_z)
        nbr_x = (1 - my_x, my_y, my_z)
        nbr_z = (my_x, my_y, 1 - my_z)

        c0 = (my_x * 2 + my_z) * CH
        cx0 = ((1 - my_x) * 2 + my_z) * CH
        cz0 = (my_x * 2 + (1 - my_z)) * CH
        cd0 = ((1 - my_x) * 2 + (1 - my_z)) * CH

        barrier_sem = pltpu.get_barrier_semaphore()
        for nbr in (nbr_x, nbr_y, nbr_z):
            pl.semaphore_signal(
                barrier_sem, inc=1, device_id=nbr, device_id_type=_MESH
            )
        pl.semaphore_wait(barrier_sem, 3)

        def rcopy(src, dst, ssem, rsem_, nbr):
            return pltpu.make_async_remote_copy(
                src_ref=src, dst_ref=dst, send_sem=ssem, recv_sem=rsem_,
                device_id=nbr, device_id_type=_MESH,
            )

        y_rdmas = []
        for b in range(NB):
            r = rcopy(
                pbf_ref.at[pl.ds(b * BR, BR)], recv_y.at[b],
                ysend.at[b], yrecv.at[b], nbr_y,
            )
            r.start()
            y_rdmas.append(r)

        gdma = pltpu.make_async_copy(g_ref, gamma_v, gsem)
        gdma.start()
        gdma.wait()

        NT = CH // TR

        def start_pr(t):
            slot = t % 2
            r0 = c0 + t * TR
            pd = pltpu.make_async_copy(
                p_ref.at[0, pl.ds(r0, TR)], pstage.at[slot], psem.at[slot]
            )
            rd = pltpu.make_async_copy(
                res_ref.at[pl.ds(r0, TR)], rstage.at[slot], rsem.at[slot]
            )
            pd.start()
            rd.start()
            return pd, rd

        pr = start_pr(0)
        x_rdmas, z_rdmas = [], []
        for t in range(NT):
            slot = t % 2
            b, h = t // 2, t % 2
            pr_next = start_pr(t + 1) if t + 1 < NT else None
            if h == 0:
                y_rdmas[b].wait_recv()
            pr[0].wait()
            pr[1].wait()
            y32 = (
                pstage[slot]
                + recv_y[b, h * TR:(h + 1) * TR, :].astype(jnp.float32)
                + rstage[slot]
            )
            ms = jnp.mean(y32 * y32, axis=-1, keepdims=True)
            norm = y32 * lax.rsqrt(ms + EPS) * gamma_v[...]
            cvt_out[slot] = norm
            ntile[slot] = norm.astype(jnp.bfloat16)
            od = pltpu.make_async_copy(
                cvt_out.at[slot], out_ref.at[pl.ds(c0 + t * TR, TR)], osem.at[slot]
            )
            nd = pltpu.make_async_copy(
                ntile.at[slot], gbuf_ref.at[pl.ds(c0 + t * TR, TR)], nsem.at[slot]
            )
            od.start()
            nd.start()
            od.wait()
            nd.wait()
            if h == 1:
                rx = rcopy(
                    gbuf_ref.at[pl.ds(c0 + b * BR, BR)],
                    gbuf_ref.at[pl.ds(c0 + b * BR, BR)],
                    xs.at[b], xr.at[b], nbr_x,
                )
                rz = rcopy(
                    gbuf_ref.at[pl.ds(c0 + b * BR, BR)],
                    gbuf_ref.at[pl.ds(c0 + b * BR, BR)],
                    zs.at[b], zr.at[b], nbr_z,
                )
                rx.start()
                rz.start()
                x_rdmas.append(rx)
                z_rdmas.append(rz)
            pr = pr_next

        cvt_queue = []

        def convert_some(n):
            for _ in range(n):
                if not cvt_queue:
                    return
                r0 = cvt_queue.pop(0)
                ci = pltpu.make_async_copy(
                    gbuf_ref.at[pl.ds(r0, TR)], cvt_in.at[0], cisem.at[0]
                )
                ci.start()
                ci.wait()
                cvt_out[0] = cvt_in[0].astype(jnp.float32)
                co = pltpu.make_async_copy(
                    cvt_out.at[0], out_ref.at[pl.ds(r0, TR)], osem.at[0]
                )
                co.start()
                co.wait()

        fx_rdmas, fz_rdmas = [], []
        for b in range(NB):
            rcopy(
                gbuf_ref.at[pl.ds(cz0 + b * BR, BR)],
                gbuf_ref.at[pl.ds(cz0 + b * BR, BR)],
                zs.at[b], zr.at[b], nbr_z,
            ).wait_recv()
            fx = rcopy(
                gbuf_ref.at[pl.ds(cz0 + b * BR, H2)],
                gbuf_ref.at[pl.ds(cz0 + b * BR, H2)],
                fxs.at[b], fxr.at[b], nbr_x,
            )
            fx.start()
            fx_rdmas.append(fx)
            rcopy(
                gbuf_ref.at[pl.ds(cx0 + b * BR, BR)],
                gbuf_ref.at[pl.ds(cx0 + b * BR, BR)],
                xs.at[b], xr.at[b], nbr_x,
            ).wait_recv()
            fz = rcopy(
                gbuf_ref.at[pl.ds(cx0 + b * BR + H2, H2)],
                gbuf_ref.at[pl.ds(cx0 + b * BR + H2, H2)],
                fzs.at[b], fzr.at[b], nbr_z,
            )
            fz.start()
            fz_rdmas.append(fz)
            cvt_queue.append(cz0 + b * BR)
            cvt_queue.append(cz0 + b * BR + TR)
            cvt_queue.append(cx0 + b * BR)
            cvt_queue.append(cx0 + b * BR + TR)
            convert_some(2)

        for b in range(NB):
            rcopy(
                gbuf_ref.at[pl.ds(cd0 + b * BR, H2)],
                gbuf_ref.at[pl.ds(cd0 + b * BR, H2)],
                fxs.at[b], fxr.at[b], nbr_x,
            ).wait_recv()
            rcopy(
                gbuf_ref.at[pl.ds(cd0 + b * BR + H2, H2)],
                gbuf_ref.at[pl.ds(cd0 + b * BR + H2, H2)],
                fzs.at[b], fzr.at[b], nbr_z,
            ).wait_recv()
            cvt_queue.append(cd0 + b * BR)
            cvt_queue.append(cd0 + b * BR + TR)
            convert_some(2)
        convert_some(len(cvt_queue))

        for r in y_rdmas + x_rdmas + z_rdmas + fx_rdmas + fz_rdmas:
            r.wait_send()

    out_f32, _gbuf = pl.pallas_call(
        body,
        out_shape=[
            jax.ShapeDtypeStruct((M, D), jnp.float32),
            jax.ShapeDtypeStruct((M, D), jnp.bfloat16),
        ],
        in_specs=[pl.BlockSpec(memory_space=pl.ANY)] * 4,
        out_specs=[pl.BlockSpec(memory_space=pl.ANY)] * 2,
        scratch_shapes=[
            pltpu.VMEM((NB, BR, D), jnp.bfloat16),
            pltpu.VMEM((1, D), jnp.float32),
            pltpu.VMEM((2, TR, D), jnp.float32),
            pltpu.VMEM((2, TR, D), jnp.float32),
            pltpu.VMEM((2, TR, D), jnp.bfloat16),
            pltpu.VMEM((2, TR, D), jnp.bfloat16),
            pltpu.VMEM((2, TR, D), jnp.float32),
        ]
        + [pltpu.SemaphoreType.DMA((NB,))] * 10
        + [
            pltpu.SemaphoreType.DMA,
            pltpu.SemaphoreType.DMA((2,)),
            pltpu.SemaphoreType.DMA((2,)),
            pltpu.SemaphoreType.DMA((2,)),
            pltpu.SemaphoreType.DMA((2,)),
            pltpu.SemaphoreType.DMA((2,)),
        ],
        compiler_params=pltpu.CompilerParams(collective_id=0),
    )(partial, pbf_c, resid, gamma2d)
    return out_f32


def kernel(partial, resid, gamma):
    my_x = lax.axis_index("x")
    my_z = lax.axis_index("z")
    c = (my_x * 2 + my_z).astype(jnp.int32).reshape((1,))

    pbf_c = _to_bf16_chunk(partial, c)
    return _fused(partial, pbf_c, resid, gamma.reshape(1, D))


# device time: 228436 ns/iter; 3.5345x vs baseline; 1.0996x over previous
import jax
import jax.numpy as jnp
from jax import lax
from jax.experimental import pallas as pl
from jax.experimental.pallas import tpu as pltpu

M = 4096
D = 4096
CH = M // 4
NB = 4
BR = CH // NB
TR = BR // 2
H2 = BR // 2
TILE = 128
EPS = 1e-6

_MESH = pl.DeviceIdType.MESH


def _to_bf16_chunk(partial, c):

    def body(c_ref, p_ref, o_ref):
        o_ref[...] = p_ref[0].astype(jnp.bfloat16)

    grid_spec = pltpu.PrefetchScalarGridSpec(
        num_scalar_prefetch=1,
        grid=(CH // TILE,),
        in_specs=[
            pl.BlockSpec((1, TILE, D), lambda i, c: (0, c[0] * (CH // TILE) + i, 0)),
        ],
        out_specs=pl.BlockSpec((TILE, D), lambda i, c: (i, 0)),
    )
    return pl.pallas_call(
        body,
        grid_spec=grid_spec,
        out_shape=jax.ShapeDtypeStruct((CH, D), jnp.bfloat16),
    )(c, partial)


def _fused(partial, pbf_c, resid, gamma2d):
    def body(
        p_ref, pbf_ref, res_ref, g_ref,
        out_ref, gbuf_ref,
        recv_y, gamma_v, pstage, rstage, ntile, cvt_in, cvt_out,
        ysend, yrecv, xs, xr, zs, zr, fxs, fxr, fzs, fzr,
        gsem, psem, rsem, nsem, osem, cisem,
    ):
        my_x = lax.axis_index("x")
        my_y = lax.axis_index("y")
        my_z = lax.axis_index("z")
        nbr_y = (my_x, 1 - my_y, my_z)
        nbr_x = (1 - my_x, my_y, my_z)
        nbr_z = (my_x, my_y, 1 - my_z)

        c0 = (my_x * 2 + my_z) * CH
        cx0 = ((1 - my_x) * 2 + my_z) * CH
        cz0 = (my_x * 2 + (1 - my_z)) * CH
        cd0 = ((1 - my_x) * 2 + (1 - my_z)) * CH

        barrier_sem = pltpu.get_barrier_semaphore()
        for nbr in (nbr_x, nbr_y, nbr_z):
            pl.semaphore_signal(
                barrier_sem, inc=1, device_id=nbr, device_id_type=_MESH
            )
        pl.semaphore_wait(barrier_sem, 3)

        def rcopy(src, dst, ssem, rsem_, nbr):
            return pltpu.make_async_remote_copy(
                src_ref=src, dst_ref=dst, send_sem=ssem, recv_sem=rsem_,
                device_id=nbr, device_id_type=_MESH,
            )

        y_rdmas = []
        for b in range(NB):
            r = rcopy(
                pbf_ref.at[pl.ds(b * BR, BR)], recv_y.at[b],
                ysend.at[b], yrecv.at[b], nbr_y,
            )
            r.start()
            y_rdmas.append(r)

        gdma = pltpu.make_async_copy(g_ref, gamma_v, gsem)
        gdma.start()
        gdma.wait()

        NT = CH // TR

        def start_pr(t):
            slot = t % 2
            r0 = c0 + t * TR
            pd = pltpu.make_async_copy(
                p_ref.at[0, pl.ds(r0, TR)], pstage.at[slot], psem.at[slot]
            )
            rd = pltpu.make_async_copy(
                res_ref.at[pl.ds(r0, TR)], rstage.at[slot], rsem.at[slot]
            )
            pd.start()
            rd.start()
            return pd, rd

        pr = start_pr(0)
        x_rdmas, z_rdmas = [], []
        for t in range(NT):
            slot = t % 2
            b, h = t // 2, t % 2
            pr_next = start_pr(t + 1) if t + 1 < NT else None
            if h == 0:
                y_rdmas[b].wait_recv()
            pr[0].wait()
            pr[1].wait()
            y32 = (
                pstage[slot]
                + recv_y[b, h * TR:(h + 1) * TR, :].astype(jnp.float32)
                + rstage[slot]
            )
            ms = jnp.mean(y32 * y32, axis=-1, keepdims=True)
            norm = y32 * lax.rsqrt(ms + EPS) * gamma_v[...]
            cvt_out[slot] = norm
            ntile[slot] = norm.astype(jnp.bfloat16)
            od = pltpu.make_async_copy(
                cvt_out.at[slot], out_ref.at[pl.ds(c0 + t * TR, TR)], osem.at[slot]
            )
            nd = pltpu.make_async_copy(
                ntile.at[slot], gbuf_ref.at[pl.ds(c0 + t * TR, TR)], nsem.at[slot]
            )
            od.start()
            nd.start()
            od.wait()
            nd.wait()
            if h == 1:
                rx = rcopy(
                    gbuf_ref.at[pl.ds(c0 + b * BR, BR)],
                    gbuf_ref.at[pl.ds(c0 + b * BR, BR)],
                    xs.at[b], xr.at[b], nbr_x,
                )
                rz = rcopy(
                    gbuf_ref.at[pl.ds(c0 + b * BR, BR)],
                    gbuf_ref.at[pl.ds(c0 + b * BR, BR)],
                    zs.at[b], zr.at[b], nbr_z,
                )
                rx.start()
                rz.start()
                x_rdmas.append(rx)
                z_rdmas.append(rz)
            pr = pr_next

        cvt_queue = []
        cvt_state = {"in": [None, None], "out": [None, None], "slot": 0}

        def _cvt_drain_out(slot):
            if cvt_state["out"][slot] is not None:
                cvt_state["out"][slot].wait()
                cvt_state["out"][slot] = None

        def _cvt_process(slot):
            if cvt_state["in"][slot] is None:
                return
            desc, r0 = cvt_state["in"][slot]
            desc.wait()
            cvt_state["in"][slot] = None
            cvt_out[slot] = cvt_in[slot].astype(jnp.float32)
            od = pltpu.make_async_copy(
                cvt_out.at[slot], out_ref.at[pl.ds(r0, TR)], osem.at[slot]
            )
            od.start()
            cvt_state["out"][slot] = od

        def convert_some(n):
            for _ in range(n):
                if not cvt_queue:
                    return
                r0 = cvt_queue.pop(0)
                slot = cvt_state["slot"]
                cvt_state["slot"] = 1 - slot
                _cvt_process(slot)
                _cvt_drain_out(slot)
                ci = pltpu.make_async_copy(
                    gbuf_ref.at[pl.ds(r0, TR)], cvt_in.at[slot], cisem.at[slot]
                )
                ci.start()
                cvt_state["in"][slot] = (ci, r0)
                _cvt_process(1 - slot)

        def convert_flush():
            convert_some(len(cvt_queue))
            for slot in (0, 1):
                _cvt_process(slot)
            for slot in (0, 1):
                _cvt_drain_out(slot)

        fx_rdmas, fz_rdmas = [], []
        for b in range(NB):
            rcopy(
                gbuf_ref.at[pl.ds(cz0 + b * BR, BR)],
                gbuf_ref.at[pl.ds(cz0 + b * BR, BR)],
                zs.at[b], zr.at[b], nbr_z,
            ).wait_recv()
            fx = rcopy(
                gbuf_ref.at[pl.ds(cz0 + b * BR, H2)],
                gbuf_ref.at[pl.ds(cz0 + b * BR, H2)],
                fxs.at[b], fxr.at[b], nbr_x,
            )
            fx.start()
            fx_rdmas.append(fx)
            rcopy(
                gbuf_ref.at[pl.ds(cx0 + b * BR, BR)],
                gbuf_ref.at[pl.ds(cx0 + b * BR, BR)],
                xs.at[b], xr.at[b], nbr_x,
            ).wait_recv()
            fz = rcopy(
                gbuf_ref.at[pl.ds(cx0 + b * BR + H2, H2)],
                gbuf_ref.at[pl.ds(cx0 + b * BR + H2, H2)],
                fzs.at[b], fzr.at[b], nbr_z,
            )
            fz.start()
            fz_rdmas.append(fz)
            cvt_queue.append(cz0 + b * BR)
            cvt_queue.append(cz0 + b * BR + TR)
            cvt_queue.append(cx0 + b * BR)
            cvt_queue.append(cx0 + b * BR + TR)
            convert_some(4)

        for b in range(NB):
            rcopy(
                gbuf_ref.at[pl.ds(cd0 + b * BR, H2)],
                gbuf_ref.at[pl.ds(cd0 + b * BR, H2)],
                fxs.at[b], fxr.at[b], nbr_x,
            ).wait_recv()
            rcopy(
                gbuf_ref.at[pl.ds(cd0 + b * BR + H2, H2)],
                gbuf_ref.at[pl.ds(cd0 + b * BR + H2, H2)],
                fzs.at[b], fzr.at[b], nbr_z,
            ).wait_recv()
            cvt_queue.append(cd0 + b * BR)
            cvt_queue.append(cd0 + b * BR + TR)
            convert_some(2)
        convert_flush()

        for r in y_rdmas + x_rdmas + z_rdmas + fx_rdmas + fz_rdmas:
            r.wait_send()

    out_f32, _gbuf = pl.pallas_call(
        body,
        out_shape=[
            jax.ShapeDtypeStruct((M, D), jnp.float32),
            jax.ShapeDtypeStruct((M, D), jnp.bfloat16),
        ],
        in_specs=[pl.BlockSpec(memory_space=pl.ANY)] * 4,
        out_specs=[pl.BlockSpec(memory_space=pl.ANY)] * 2,
        scratch_shapes=[
            pltpu.VMEM((NB, BR, D), jnp.bfloat16),
            pltpu.VMEM((1, D), jnp.float32),
            pltpu.VMEM((2, TR, D), jnp.float32),
            pltpu.VMEM((2, TR, D), jnp.float32),
            pltpu.VMEM((2, TR, D), jnp.bfloat16),
            pltpu.VMEM((2, TR, D), jnp.bfloat16),
            pltpu.VMEM((2, TR, D), jnp.float32),
        ]
        + [pltpu.SemaphoreType.DMA((NB,))] * 10
        + [
            pltpu.SemaphoreType.DMA,
            pltpu.SemaphoreType.DMA((2,)),
            pltpu.SemaphoreType.DMA((2,)),
            pltpu.SemaphoreType.DMA((2,)),
            pltpu.SemaphoreType.DMA((2,)),
            pltpu.SemaphoreType.DMA((2,)),
        ],
        compiler_params=pltpu.CompilerParams(collective_id=0),
    )(partial, pbf_c, resid, gamma2d)
    return out_f32


def kernel(partial, resid, gamma):
    my_x = lax.axis_index("x")
    my_z = lax.axis_index("z")
    c = (my_x * 2 + my_z).astype(jnp.int32).reshape((1,))

    pbf_c = _to_bf16_chunk(partial, c)
    return _fused(partial, pbf_c, resid, gamma.reshape(1, D))


# device time: 214111 ns/iter; 3.7710x vs baseline; 1.0669x over previous
import jax
import jax.numpy as jnp
from jax import lax
from jax.experimental import pallas as pl
from jax.experimental.pallas import tpu as pltpu

M = 4096
D = 4096
CH = M // 4
NB = 8
BR = CH // NB
TR = BR
H2 = BR // 2
TILE = 128
EPS = 1e-6

_MESH = pl.DeviceIdType.MESH


def _to_bf16_chunk(partial, c):

    def body(c_ref, p_ref, o_ref):
        o_ref[...] = p_ref[0].astype(jnp.bfloat16)

    grid_spec = pltpu.PrefetchScalarGridSpec(
        num_scalar_prefetch=1,
        grid=(CH // TILE,),
        in_specs=[
            pl.BlockSpec((1, TILE, D), lambda i, c: (0, c[0] * (CH // TILE) + i, 0)),
        ],
        out_specs=pl.BlockSpec((TILE, D), lambda i, c: (i, 0)),
    )
    return pl.pallas_call(
        body,
        grid_spec=grid_spec,
        out_shape=jax.ShapeDtypeStruct((CH, D), jnp.bfloat16),
    )(c, partial)


def _fused(partial, pbf_c, resid, gamma2d):
    def body(
        p_ref, pbf_ref, res_ref, g_ref,
        out_ref, gbuf_ref,
        recv_y, gamma_v, pstage, rstage, ntile, cvt_in, cvt_out,
        ysend, yrecv, xs, xr, zs, zr, fxs, fxr, fzs, fzr,
        gsem, psem, rsem, nsem, osem, cisem,
    ):
        my_x = lax.axis_index("x")
        my_y = lax.axis_index("y")
        my_z = lax.axis_index("z")
        nbr_y = (my_x, 1 - my_y, my_z)
        nbr_x = (1 - my_x, my_y, my_z)
        nbr_z = (my_x, my_y, 1 - my_z)

        c0 = (my_x * 2 + my_z) * CH
        cx0 = ((1 - my_x) * 2 + my_z) * CH
        cz0 = (my_x * 2 + (1 - my_z)) * CH
        cd0 = ((1 - my_x) * 2 + (1 - my_z)) * CH

        barrier_sem = pltpu.get_barrier_semaphore()
        for nbr in (nbr_x, nbr_y, nbr_z):
            pl.semaphore_signal(
                barrier_sem, inc=1, device_id=nbr, device_id_type=_MESH
            )
        pl.semaphore_wait(barrier_sem, 3)

        def rcopy(src, dst, ssem, rsem_, nbr):
            return pltpu.make_async_remote_copy(
                src_ref=src, dst_ref=dst, send_sem=ssem, recv_sem=rsem_,
                device_id=nbr, device_id_type=_MESH,
            )

        y_rdmas = []
        for b in range(NB):
            r = rcopy(
                pbf_ref.at[pl.ds(b * BR, BR)], recv_y.at[b],
                ysend.at[b], yrecv.at[b], nbr_y,
            )
            r.start()
            y_rdmas.append(r)

        gdma = pltpu.make_async_copy(g_ref, gamma_v, gsem)
        gdma.start()
        gdma.wait()

        def start_pr(b):
            slot = b % 2
            r0 = c0 + b * BR
            pd = pltpu.make_async_copy(
                p_ref.at[0, pl.ds(r0, BR)], pstage.at[slot], psem.at[slot]
            )
            rd = pltpu.make_async_copy(
                res_ref.at[pl.ds(r0, BR)], rstage.at[slot], rsem.at[slot]
            )
            pd.start()
            rd.start()
            return pd, rd

        pr = start_pr(0)
        x_rdmas, z_rdmas = [], []
        for b in range(NB):
            slot = b % 2
            pr_next = start_pr(b + 1) if b + 1 < NB else None
            y_rdmas[b].wait_recv()
            pr[0].wait()
            pr[1].wait()
            y32 = (
                pstage[slot]
                + recv_y[b].astype(jnp.float32)
                + rstage[slot]
            )
            ms = jnp.mean(y32 * y32, axis=-1, keepdims=True)
            norm = y32 * lax.rsqrt(ms + EPS) * gamma_v[...]
            cvt_out[slot] = norm
            ntile[slot] = norm.astype(jnp.bfloat16)
            od = pltpu.make_async_copy(
                cvt_out.at[slot], out_ref.at[pl.ds(c0 + b * BR, BR)], osem.at[slot]
            )
            nd = pltpu.make_async_copy(
                ntile.at[slot], gbuf_ref.at[pl.ds(c0 + b * BR, BR)], nsem.at[slot]
            )
            od.start()
            nd.start()
            od.wait()
            nd.wait()
            rx = rcopy(
                gbuf_ref.at[pl.ds(c0 + b * BR, BR)],
                gbuf_ref.at[pl.ds(c0 + b * BR, BR)],
                xs.at[b], xr.at[b], nbr_x,
            )
            rz = rcopy(
                gbuf_ref.at[pl.ds(c0 + b * BR, BR)],
                gbuf_ref.at[pl.ds(c0 + b * BR, BR)],
                zs.at[b], zr.at[b], nbr_z,
            )
            rx.start()
            rz.start()
            x_rdmas.append(rx)
            z_rdmas.append(rz)
            pr = pr_next

        cvt_queue = []
        cvt_state = {"in": [None, None], "out": [None, None], "slot": 0}

        def _cvt_drain_out(slot):
            if cvt_state["out"][slot] is not None:
                cvt_state["out"][slot].wait()
                cvt_state["out"][slot] = None

        def _cvt_process(slot):
            if cvt_state["in"][slot] is None:
                return
            desc, r0 = cvt_state["in"][slot]
            desc.wait()
            cvt_state["in"][slot] = None
            cvt_out[slot] = cvt_in[slot].astype(jnp.float32)
            od = pltpu.make_async_copy(
                cvt_out.at[slot], out_ref.at[pl.ds(r0, TR)], osem.at[slot]
            )
            od.start()
            cvt_state["out"][slot] = od

        def convert_some(n):
            for _ in range(n):
                if not cvt_queue:
                    return
                r0 = cvt_queue.pop(0)
                slot = cvt_state["slot"]
                cvt_state["slot"] = 1 - slot
                _cvt_process(slot)
                _cvt_drain_out(slot)
                ci = pltpu.make_async_copy(
                    gbuf_ref.at[pl.ds(r0, TR)], cvt_in.at[slot], cisem.at[slot]
                )
                ci.start()
                cvt_state["in"][slot] = (ci, r0)
                _cvt_process(1 - slot)

        def convert_flush():
            convert_some(len(cvt_queue))
            for slot in (0, 1):
                _cvt_process(slot)
            for slot in (0, 1):
                _cvt_drain_out(slot)

        fx_rdmas, fz_rdmas = [], []
        for b in range(NB):
            rcopy(
                gbuf_ref.at[pl.ds(cz0 + b * BR, BR)],
                gbuf_ref.at[pl.ds(cz0 + b * BR, BR)],
                zs.at[b], zr.at[b], nbr_z,
            ).wait_recv()
            fx = rcopy(
                gbuf_ref.at[pl.ds(cz0 + b * BR, H2)],
                gbuf_ref.at[pl.ds(cz0 + b * BR, H2)],
                fxs.at[b], fxr.at[b], nbr_x,
            )
            fx.start()
            fx_rdmas.append(fx)
            rcopy(
                gbuf_ref.at[pl.ds(cx0 + b * BR, BR)],
                gbuf_ref.at[pl.ds(cx0 + b * BR, BR)],
                xs.at[b], xr.at[b], nbr_x,
            ).wait_recv()
            fz = rcopy(
                gbuf_ref.at[pl.ds(cx0 + b * BR + H2, H2)],
                gbuf_ref.at[pl.ds(cx0 + b * BR + H2, H2)],
                fzs.at[b], fzr.at[b], nbr_z,
            )
            fz.start()
            fz_rdmas.append(fz)
            cvt_queue.append(cz0 + b * BR)
            cvt_queue.append(cx0 + b * BR)
            convert_some(2)

        for b in range(NB):
            rcopy(
                gbuf_ref.at[pl.ds(cd0 + b * BR, H2)],
                gbuf_ref.at[pl.ds(cd0 + b * BR, H2)],
                fxs.at[b], fxr.at[b], nbr_x,
            ).wait_recv()
            rcopy(
                gbuf_ref.at[pl.ds(cd0 + b * BR + H2, H2)],
                gbuf_ref.at[pl.ds(cd0 + b * BR + H2, H2)],
                fzs.at[b], fzr.at[b], nbr_z,
            ).wait_recv()
            cvt_queue.append(cd0 + b * BR)
            convert_some(1)
        convert_flush()

        for r in y_rdmas + x_rdmas + z_rdmas + fx_rdmas + fz_rdmas:
            r.wait_send()

    out_f32, _gbuf = pl.pallas_call(
        body,
        out_shape=[
            jax.ShapeDtypeStruct((M, D), jnp.float32),
            jax.ShapeDtypeStruct((M, D), jnp.bfloat16),
        ],
        in_specs=[pl.BlockSpec(memory_space=pl.ANY)] * 4,
        out_specs=[pl.BlockSpec(memory_space=pl.ANY)] * 2,
        scratch_shapes=[
            pltpu.VMEM((NB, BR, D), jnp.bfloat16),
            pltpu.VMEM((1, D), jnp.float32),
            pltpu.VMEM((2, TR, D), jnp.float32),
            pltpu.VMEM((2, TR, D), jnp.float32),
            pltpu.VMEM((2, TR, D), jnp.bfloat16),
            pltpu.VMEM((2, TR, D), jnp.bfloat16),
            pltpu.VMEM((2, TR, D), jnp.float32),
        ]
        + [pltpu.SemaphoreType.DMA((NB,))] * 10
        + [
            pltpu.SemaphoreType.DMA,
            pltpu.SemaphoreType.DMA((2,)),
            pltpu.SemaphoreType.DMA((2,)),
            pltpu.SemaphoreType.DMA((2,)),
            pltpu.SemaphoreType.DMA((2,)),
            pltpu.SemaphoreType.DMA((2,)),
        ],
        compiler_params=pltpu.CompilerParams(collective_id=0),
    )(partial, pbf_c, resid, gamma2d)
    return out_f32


def kernel(partial, resid, gamma):
    my_x = lax.axis_index("x")
    my_z = lax.axis_index("z")
    c = (my_x * 2 + my_z).astype(jnp.int32).reshape((1,))

    pbf_c = _to_bf16_chunk(partial, c)
    return _fused(partial, pbf_c, resid, gamma.reshape(1, D))
